# Initial kernel scaffold; baseline (speedup 1.0000x reference)
import jax
import jax.numpy as jnp
from jax.experimental import pallas as pl


def kernel(x, patch_W, patch_b, pos_embed, router_W, ln_g, ln_b, W1, b1, W2, b2, head_W):
    raise NotImplementedError("write your pallas kernel here")


# speedup vs baseline: 3.1527x; 3.1527x over previous
import functools

import jax
import jax.numpy as jnp
from jax.experimental import pallas as pl
from jax.experimental.pallas import tpu as pltpu

_B = 64
_C_IN = 3
_IMG = 224
_PATCH = 16
_NP = (_IMG // _PATCH) ** 2
_D = 768
_D_FF = 3072
_M = 8
_K = 2
_HOPS = 4
_NCLS = 1000
_CPP = _C_IN * _PATCH * _PATCH
_A = _B * _K
_EPS = 1e-6


def _patch_body(xp_ref, w_ref, b_ref, pos_ref, rw_ref, h_ref, log_ref):
    t = xp_ref[0]
    hb = jnp.dot(t, w_ref[...], preferred_element_type=jnp.float32)
    hb = hb + b_ref[0] + pos_ref[0]
    h_ref[0] = hb
    hbar = jnp.mean(hb, axis=0, keepdims=True)
    log_ref[0] = jnp.dot(hbar, rw_ref[...], preferred_element_type=jnp.float32)


def _patch_embed(xp, patch_W, patch_b2, pos_embed, rw0):
    return pl.pallas_call(
        _patch_body,
        grid=(_B,),
        in_specs=[
            pl.BlockSpec((1, _NP, _CPP), lambda b: (b, 0, 0)),
            pl.BlockSpec((_CPP, _D), lambda b: (0, 0)),
            pl.BlockSpec((1, _D), lambda b: (0, 0)),
            pl.BlockSpec((1, _NP, _D), lambda b: (0, 0, 0)),
            pl.BlockSpec((_D, _M), lambda b: (0, 0)),
        ],
        out_specs=[
            pl.BlockSpec((1, _NP, _D), lambda b: (b, 0, 0)),
            pl.BlockSpec((1, 1, _M), lambda b: (b, 0, 0)),
        ],
        out_shape=[
            jax.ShapeDtypeStruct((_B, _NP, _D), jnp.float32),
            jax.ShapeDtypeStruct((_B, 1, _M), jnp.float32),
        ],
    )(xp, patch_W, patch_b2, pos_embed, rw0)


def _expert_body(bs_ref, es_ref, h_ref, lg_ref, lb_ref, w1_ref, b1_ref,
                 w2_ref, b2_ref, out_ref):
    x = h_ref[0]
    mu = jnp.mean(x, axis=-1, keepdims=True)
    xc = x - mu
    var = jnp.mean(xc * xc, axis=-1, keepdims=True)
    xln = xc * jax.lax.rsqrt(var + _EPS) * lg_ref[0] + lb_ref[0]
    h1 = jnp.dot(xln, w1_ref[0], preferred_element_type=jnp.float32) + b1_ref[0]
    h1 = jax.nn.gelu(h1)
    y = jnp.dot(h1, w2_ref[0], preferred_element_type=jnp.float32) + b2_ref[0]
    out_ref[0] = y


def _expert_mlp(b_sorted, e_sorted, h, ln_g3, ln_b3, W1, b13, W2, b23):
    grid_spec = pltpu.PrefetchScalarGridSpec(
        num_scalar_prefetch=2,
        grid=(_A,),
        in_specs=[
            pl.BlockSpec((1, _NP, _D), lambda a, bs, es: (bs[a], 0, 0)),
            pl.BlockSpec((1, 1, _D), lambda a, bs, es: (es[a], 0, 0)),
            pl.BlockSpec((1, 1, _D), lambda a, bs, es: (es[a], 0, 0)),
            pl.BlockSpec((1, _D, _D_FF), lambda a, bs, es: (es[a], 0, 0)),
            pl.BlockSpec((1, 1, _D_FF), lambda a, bs, es: (es[a], 0, 0)),
            pl.BlockSpec((1, _D_FF, _D), lambda a, bs, es: (es[a], 0, 0)),
            pl.BlockSpec((1, 1, _D), lambda a, bs, es: (es[a], 0, 0)),
        ],
        out_specs=pl.BlockSpec((1, _NP, _D), lambda a, bs, es: (a, 0, 0)),
    )
    return pl.pallas_call(
        _expert_body,
        grid_spec=grid_spec,
        out_shape=jax.ShapeDtypeStruct((_A, _NP, _D), jnp.float32),
    )(b_sorted, e_sorted, h, ln_g3, ln_b3, W1, b13, W2, b23)


def _combine_body(p0_ref, p1_ref, h_ref, o0_ref, o1_ref, w0_ref, w1_ref,
                  rw_ref, h_out, log_ref):
    w0 = w0_ref[0, 0, 0]
    w1 = w1_ref[0, 0, 0]
    hn = (w0 + w1) * h_ref[0] + w0 * o0_ref[0] + w1 * o1_ref[0]
    h_out[0] = hn
    hbar = jnp.mean(hn, axis=0, keepdims=True)
    log_ref[0] = jnp.dot(hbar, rw_ref[...], preferred_element_type=jnp.float32)


def _combine(pos0, pos1, h, outbuf, w0, w1, rw_next):
    grid_spec = pltpu.PrefetchScalarGridSpec(
        num_scalar_prefetch=2,
        grid=(_B,),
        in_specs=[
            pl.BlockSpec((1, _NP, _D), lambda b, p0, p1: (b, 0, 0)),
            pl.BlockSpec((1, _NP, _D), lambda b, p0, p1: (p0[b], 0, 0)),
            pl.BlockSpec((1, _NP, _D), lambda b, p0, p1: (p1[b], 0, 0)),
            pl.BlockSpec((1, 1, 1), lambda b, p0, p1: (b, 0, 0)),
            pl.BlockSpec((1, 1, 1), lambda b, p0, p1: (b, 0, 0)),
            pl.BlockSpec((_D, _M), lambda b, p0, p1: (0, 0)),
        ],
        out_specs=[
            pl.BlockSpec((1, _NP, _D), lambda b, p0, p1: (b, 0, 0)),
            pl.BlockSpec((1, 1, _M), lambda b, p0, p1: (b, 0, 0)),
        ],
    )
    return pl.pallas_call(
        _combine_body,
        grid_spec=grid_spec,
        out_shape=[
            jax.ShapeDtypeStruct((_B, _NP, _D), jnp.float32),
            jax.ShapeDtypeStruct((_B, 1, _M), jnp.float32),
        ],
    )(pos0, pos1, h, outbuf, outbuf, w0, w1, rw_next)


def _final_body(p0_ref, p1_ref, h_ref, o0_ref, o1_ref, w0_ref, w1_ref,
                hw_ref, log_ref):
    w0 = w0_ref[0, 0, 0]
    w1 = w1_ref[0, 0, 0]
    hn = (w0 + w1) * h_ref[0] + w0 * o0_ref[0] + w1 * o1_ref[0]
    hbar = jnp.mean(hn, axis=0, keepdims=True)
    log_ref[0] = jnp.dot(hbar, hw_ref[...], preferred_element_type=jnp.float32)


def _final_combine(pos0, pos1, h, outbuf, w0, w1, head_W):
    grid_spec = pltpu.PrefetchScalarGridSpec(
        num_scalar_prefetch=2,
        grid=(_B,),
        in_specs=[
            pl.BlockSpec((1, _NP, _D), lambda b, p0, p1: (b, 0, 0)),
            pl.BlockSpec((1, _NP, _D), lambda b, p0, p1: (p0[b], 0, 0)),
            pl.BlockSpec((1, _NP, _D), lambda b, p0, p1: (p1[b], 0, 0)),
            pl.BlockSpec((1, 1, 1), lambda b, p0, p1: (b, 0, 0)),
            pl.BlockSpec((1, 1, 1), lambda b, p0, p1: (b, 0, 0)),
            pl.BlockSpec((_D, _NCLS), lambda b, p0, p1: (0, 0)),
        ],
        out_specs=pl.BlockSpec((1, 1, _NCLS), lambda b, p0, p1: (b, 0, 0)),
    )
    return pl.pallas_call(
        _final_body,
        grid_spec=grid_spec,
        out_shape=jax.ShapeDtypeStruct((_B, 1, _NCLS), jnp.float32),
    )(pos0, pos1, h, outbuf, outbuf, w0, w1, head_W)


def _route(logits3):
    logits = logits3[:, 0, :]
    probs = jax.nn.softmax(logits, axis=-1)
    top_vals, top_idx = jax.lax.top_k(probs, _K)
    sw = top_vals / (jnp.sum(top_vals, axis=-1, keepdims=True) + 1e-6)
    e_flat = top_idx.reshape(_A).astype(jnp.int32)
    order = jnp.argsort(e_flat).astype(jnp.int32)
    b_sorted = (order // _K).astype(jnp.int32)
    e_sorted = e_flat[order]
    inv = jnp.zeros((_A,), jnp.int32).at[order].set(
        jnp.arange(_A, dtype=jnp.int32))
    pos0 = inv[0::2]
    pos1 = inv[1::2]
    w0 = sw[:, 0].reshape(_B, 1, 1)
    w1 = sw[:, 1].reshape(_B, 1, 1)
    return b_sorted, e_sorted, pos0, pos1, w0, w1


@functools.partial(jax.jit)
def kernel(x, patch_W, patch_b, pos_embed, router_W, ln_g, ln_b,
           W1, b1, W2, b2, head_W):
    g = _IMG // _PATCH
    xp = x.reshape(_B, _C_IN, g, _PATCH, g, _PATCH)
    xp = xp.transpose(0, 2, 4, 1, 3, 5).reshape(_B, _NP, _CPP)
    patch_b2 = patch_b.reshape(1, _D)
    ln_g3 = ln_g.reshape(_M, 1, _D)
    ln_b3 = ln_b.reshape(_M, 1, _D)
    b13 = b1.reshape(_M, 1, _D_FF)
    b23 = b2.reshape(_M, 1, _D)

    h, logits = _patch_embed(xp, patch_W, patch_b2, pos_embed, router_W[0])

    for hop in range(_HOPS):
        b_sorted, e_sorted, pos0, pos1, w0, w1 = _route(logits)
        outbuf = _expert_mlp(b_sorted, e_sorted, h, ln_g3, ln_b3, W1, b13,
                             W2, b23)
        if hop < _HOPS - 1:
            h, logits = _combine(pos0, pos1, h, outbuf, w0, w1,
                                 router_W[hop + 1])
        else:
            out = _final_combine(pos0, pos1, h, outbuf, w0, w1, head_W)
    return out.reshape(_B, _NCLS)


# speedup vs baseline: 3.1780x; 1.0080x over previous
import functools

import jax
import jax.numpy as jnp
from jax.experimental import pallas as pl
from jax.experimental.pallas import tpu as pltpu

_B = 64
_C_IN = 3
_IMG = 224
_PATCH = 16
_NP = (_IMG // _PATCH) ** 2
_D = 768
_D_FF = 3072
_M = 8
_K = 2
_HOPS = 4
_NCLS = 1000
_CPP = _C_IN * _PATCH * _PATCH
_A = _B * _K
_EPS = 1e-6


def _patch_body(xp_ref, w_ref, b_ref, pos_ref, rw_ref, h_ref, log_ref):
    t = xp_ref[0]
    hb = jnp.dot(t, w_ref[...], preferred_element_type=jnp.float32)
    hb = hb + b_ref[0] + pos_ref[0]
    h_ref[0] = hb
    hbar = jnp.mean(hb, axis=0, keepdims=True)
    log_ref[0] = jnp.dot(hbar, rw_ref[...], preferred_element_type=jnp.float32)


def _patch_embed(xp, patch_W, patch_b2, pos_embed, rw0):
    return pl.pallas_call(
        _patch_body,
        grid=(_B,),
        in_specs=[
            pl.BlockSpec((1, _NP, _CPP), lambda b: (b, 0, 0)),
            pl.BlockSpec((_CPP, _D), lambda b: (0, 0)),
            pl.BlockSpec((1, _D), lambda b: (0, 0)),
            pl.BlockSpec((1, _NP, _D), lambda b: (0, 0, 0)),
            pl.BlockSpec((_D, _M), lambda b: (0, 0)),
        ],
        out_specs=[
            pl.BlockSpec((1, _NP, _D), lambda b: (b, 0, 0)),
            pl.BlockSpec((1, 1, _M), lambda b: (b, 0, 0)),
        ],
        out_shape=[
            jax.ShapeDtypeStruct((_B, _NP, _D), jnp.float32),
            jax.ShapeDtypeStruct((_B, 1, _M), jnp.float32),
        ],
    )(xp, patch_W, patch_b2, pos_embed, rw0)


def _expert_body(bs_ref, es_ref, h_ref, lg_ref, lb_ref, w1_ref, b1_ref,
                 w2_ref, b2_ref, out_ref):
    x = h_ref[0]
    mu = jnp.mean(x, axis=-1, keepdims=True)
    xc = x - mu
    var = jnp.mean(xc * xc, axis=-1, keepdims=True)
    xln = xc * jax.lax.rsqrt(var + _EPS) * lg_ref[0] + lb_ref[0]
    h1 = jnp.dot(xln.astype(jnp.bfloat16), w1_ref[0],
                 preferred_element_type=jnp.float32) + b1_ref[0]
    h1 = jax.nn.gelu(h1)
    y = jnp.dot(h1.astype(jnp.bfloat16), w2_ref[0],
                preferred_element_type=jnp.float32) + b2_ref[0]
    out_ref[0] = y


def _expert_mlp(b_sorted, e_sorted, h, ln_g3, ln_b3, W1, b13, W2, b23):
    grid_spec = pltpu.PrefetchScalarGridSpec(
        num_scalar_prefetch=2,
        grid=(_A,),
        in_specs=[
            pl.BlockSpec((1, _NP, _D), lambda a, bs, es: (bs[a], 0, 0)),
            pl.BlockSpec((1, 1, _D), lambda a, bs, es: (es[a], 0, 0)),
            pl.BlockSpec((1, 1, _D), lambda a, bs, es: (es[a], 0, 0)),
            pl.BlockSpec((1, _D, _D_FF), lambda a, bs, es: (es[a], 0, 0)),
            pl.BlockSpec((1, 1, _D_FF), lambda a, bs, es: (es[a], 0, 0)),
            pl.BlockSpec((1, _D_FF, _D), lambda a, bs, es: (es[a], 0, 0)),
            pl.BlockSpec((1, 1, _D), lambda a, bs, es: (es[a], 0, 0)),
        ],
        out_specs=pl.BlockSpec((1, _NP, _D), lambda a, bs, es: (a, 0, 0)),
    )
    return pl.pallas_call(
        _expert_body,
        grid_spec=grid_spec,
        out_shape=jax.ShapeDtypeStruct((_A, _NP, _D), jnp.float32),
    )(b_sorted, e_sorted, h, ln_g3, ln_b3, W1, b13, W2, b23)


def _combine_body(p0_ref, p1_ref, h_ref, o0_ref, o1_ref, w0_ref, w1_ref,
                  rw_ref, h_out, log_ref):
    w0 = w0_ref[0, 0, 0]
    w1 = w1_ref[0, 0, 0]
    hn = (w0 + w1) * h_ref[0] + w0 * o0_ref[0] + w1 * o1_ref[0]
    h_out[0] = hn
    hbar = jnp.mean(hn, axis=0, keepdims=True)
    log_ref[0] = jnp.dot(hbar, rw_ref[...], preferred_element_type=jnp.float32)


def _combine(pos0, pos1, h, outbuf, w0, w1, rw_next):
    grid_spec = pltpu.PrefetchScalarGridSpec(
        num_scalar_prefetch=2,
        grid=(_B,),
        in_specs=[
            pl.BlockSpec((1, _NP, _D), lambda b, p0, p1: (b, 0, 0)),
            pl.BlockSpec((1, _NP, _D), lambda b, p0, p1: (p0[b], 0, 0)),
            pl.BlockSpec((1, _NP, _D), lambda b, p0, p1: (p1[b], 0, 0)),
            pl.BlockSpec((1, 1, 1), lambda b, p0, p1: (b, 0, 0)),
            pl.BlockSpec((1, 1, 1), lambda b, p0, p1: (b, 0, 0)),
            pl.BlockSpec((_D, _M), lambda b, p0, p1: (0, 0)),
        ],
        out_specs=[
            pl.BlockSpec((1, _NP, _D), lambda b, p0, p1: (b, 0, 0)),
            pl.BlockSpec((1, 1, _M), lambda b, p0, p1: (b, 0, 0)),
        ],
    )
    return pl.pallas_call(
        _combine_body,
        grid_spec=grid_spec,
        out_shape=[
            jax.ShapeDtypeStruct((_B, _NP, _D), jnp.float32),
            jax.ShapeDtypeStruct((_B, 1, _M), jnp.float32),
        ],
    )(pos0, pos1, h, outbuf, outbuf, w0, w1, rw_next)


def _final_body(p0_ref, p1_ref, h_ref, o0_ref, o1_ref, w0_ref, w1_ref,
                hw_ref, log_ref):
    w0 = w0_ref[0, 0, 0]
    w1 = w1_ref[0, 0, 0]
    hn = (w0 + w1) * h_ref[0] + w0 * o0_ref[0] + w1 * o1_ref[0]
    hbar = jnp.mean(hn, axis=0, keepdims=True)
    log_ref[0] = jnp.dot(hbar, hw_ref[...], preferred_element_type=jnp.float32)


def _final_combine(pos0, pos1, h, outbuf, w0, w1, head_W):
    grid_spec = pltpu.PrefetchScalarGridSpec(
        num_scalar_prefetch=2,
        grid=(_B,),
        in_specs=[
            pl.BlockSpec((1, _NP, _D), lambda b, p0, p1: (b, 0, 0)),
            pl.BlockSpec((1, _NP, _D), lambda b, p0, p1: (p0[b], 0, 0)),
            pl.BlockSpec((1, _NP, _D), lambda b, p0, p1: (p1[b], 0, 0)),
            pl.BlockSpec((1, 1, 1), lambda b, p0, p1: (b, 0, 0)),
            pl.BlockSpec((1, 1, 1), lambda b, p0, p1: (b, 0, 0)),
            pl.BlockSpec((_D, _NCLS), lambda b, p0, p1: (0, 0)),
        ],
        out_specs=pl.BlockSpec((1, 1, _NCLS), lambda b, p0, p1: (b, 0, 0)),
    )
    return pl.pallas_call(
        _final_body,
        grid_spec=grid_spec,
        out_shape=jax.ShapeDtypeStruct((_B, 1, _NCLS), jnp.float32),
    )(pos0, pos1, h, outbuf, outbuf, w0, w1, head_W)


def _route(logits3):
    logits = logits3[:, 0, :]
    probs = jax.nn.softmax(logits, axis=-1)
    top_vals, top_idx = jax.lax.top_k(probs, _K)
    sw = top_vals / (jnp.sum(top_vals, axis=-1, keepdims=True) + 1e-6)
    e_flat = top_idx.reshape(_A).astype(jnp.int32)
    order = jnp.argsort(e_flat).astype(jnp.int32)
    b_sorted = (order // _K).astype(jnp.int32)
    e_sorted = e_flat[order]
    inv = jnp.zeros((_A,), jnp.int32).at[order].set(
        jnp.arange(_A, dtype=jnp.int32))
    pos0 = inv[0::2]
    pos1 = inv[1::2]
    w0 = sw[:, 0].reshape(_B, 1, 1)
    w1 = sw[:, 1].reshape(_B, 1, 1)
    return b_sorted, e_sorted, pos0, pos1, w0, w1


@functools.partial(jax.jit)
def kernel(x, patch_W, patch_b, pos_embed, router_W, ln_g, ln_b,
           W1, b1, W2, b2, head_W):
    g = _IMG // _PATCH
    xp = x.reshape(_B, _C_IN, g, _PATCH, g, _PATCH)
    xp = xp.transpose(0, 2, 4, 1, 3, 5).reshape(_B, _NP, _CPP)
    patch_b2 = patch_b.reshape(1, _D)
    ln_g3 = ln_g.reshape(_M, 1, _D)
    ln_b3 = ln_b.reshape(_M, 1, _D)
    b13 = b1.reshape(_M, 1, _D_FF)
    b23 = b2.reshape(_M, 1, _D)
    W1c = W1.astype(jnp.bfloat16)
    W2c = W2.astype(jnp.bfloat16)

    h, logits = _patch_embed(xp, patch_W, patch_b2, pos_embed, router_W[0])

    for hop in range(_HOPS):
        b_sorted, e_sorted, pos0, pos1, w0, w1 = _route(logits)
        outbuf = _expert_mlp(b_sorted, e_sorted, h, ln_g3, ln_b3, W1c, b13,
                             W2c, b23)
        if hop < _HOPS - 1:
            h, logits = _combine(pos0, pos1, h, outbuf, w0, w1,
                                 router_W[hop + 1])
        else:
            out = _final_combine(pos0, pos1, h, outbuf, w0, w1, head_W)
    return out.reshape(_B, _NCLS)


# speedup vs baseline: 3.2549x; 1.0242x over previous
import functools

import jax
import jax.numpy as jnp
from jax.experimental import pallas as pl
from jax.experimental.pallas import tpu as pltpu

_B = 64
_C_IN = 3
_IMG = 224
_PATCH = 16
_NP = (_IMG // _PATCH) ** 2
_D = 768
_D_FF = 3072
_M = 8
_K = 2
_HOPS = 4
_NCLS = 1000
_CPP = _C_IN * _PATCH * _PATCH
_A = _B * _K
_EPS = 1e-6


def _patch_body(xp_ref, w_ref, b_ref, pos_ref, rw_ref, h_ref, log_ref):
    t = xp_ref[0]
    hb = jnp.dot(t, w_ref[...], preferred_element_type=jnp.float32)
    hb = hb + b_ref[0] + pos_ref[0]
    h_ref[0] = hb
    hbar = jnp.mean(hb, axis=0, keepdims=True)
    log_ref[0] = jnp.dot(hbar, rw_ref[...], preferred_element_type=jnp.float32)


def _patch_embed(xp, patch_W, patch_b2, pos_embed, rw0):
    return pl.pallas_call(
        _patch_body,
        grid=(_B,),
        in_specs=[
            pl.BlockSpec((1, _NP, _CPP), lambda b: (b, 0, 0)),
            pl.BlockSpec((_CPP, _D), lambda b: (0, 0)),
            pl.BlockSpec((1, _D), lambda b: (0, 0)),
            pl.BlockSpec((1, _NP, _D), lambda b: (0, 0, 0)),
            pl.BlockSpec((_D, _M), lambda b: (0, 0)),
        ],
        out_specs=[
            pl.BlockSpec((1, _NP, _D), lambda b: (b, 0, 0)),
            pl.BlockSpec((1, 1, _M), lambda b: (b, 0, 0)),
        ],
        out_shape=[
            jax.ShapeDtypeStruct((_B, _NP, _D), jnp.float32),
            jax.ShapeDtypeStruct((_B, 1, _M), jnp.float32),
        ],
    )(xp, patch_W, patch_b2, pos_embed, rw0)


def _expert_body(bs_ref, es_ref, h_ref, lg_ref, lb_ref, w1_ref, b1_ref,
                 w2_ref, b2_ref, out_ref):
    x = h_ref[0]
    mu = jnp.mean(x, axis=-1, keepdims=True)
    xc = x - mu
    var = jnp.mean(xc * xc, axis=-1, keepdims=True)
    xln = xc * jax.lax.rsqrt(var + _EPS) * lg_ref[0] + lb_ref[0]
    h1 = jnp.dot(xln.astype(jnp.bfloat16), w1_ref[0],
                 preferred_element_type=jnp.float32)
    h1 = jax.nn.gelu((h1 + b1_ref[0]).astype(jnp.bfloat16))
    y = jnp.dot(h1, w2_ref[0],
                preferred_element_type=jnp.float32) + b2_ref[0]
    out_ref[0] = y.astype(jnp.bfloat16)


def _expert_mlp(b_sorted, e_sorted, h, ln_g3, ln_b3, W1, b13, W2, b23):
    grid_spec = pltpu.PrefetchScalarGridSpec(
        num_scalar_prefetch=2,
        grid=(_A,),
        in_specs=[
            pl.BlockSpec((1, _NP, _D), lambda a, bs, es: (bs[a], 0, 0)),
            pl.BlockSpec((1, 1, _D), lambda a, bs, es: (es[a], 0, 0)),
            pl.BlockSpec((1, 1, _D), lambda a, bs, es: (es[a], 0, 0)),
            pl.BlockSpec((1, _D, _D_FF), lambda a, bs, es: (es[a], 0, 0)),
            pl.BlockSpec((1, 1, _D_FF), lambda a, bs, es: (es[a], 0, 0)),
            pl.BlockSpec((1, _D_FF, _D), lambda a, bs, es: (es[a], 0, 0)),
            pl.BlockSpec((1, 1, _D), lambda a, bs, es: (es[a], 0, 0)),
        ],
        out_specs=pl.BlockSpec((1, _NP, _D), lambda a, bs, es: (a, 0, 0)),
    )
    return pl.pallas_call(
        _expert_body,
        grid_spec=grid_spec,
        out_shape=jax.ShapeDtypeStruct((_A, _NP, _D), jnp.bfloat16),
    )(b_sorted, e_sorted, h, ln_g3, ln_b3, W1, b13, W2, b23)


def _combine_body(p0_ref, p1_ref, h_ref, o0_ref, o1_ref, w0_ref, w1_ref,
                  rw_ref, h_out, log_ref):
    w0 = w0_ref[0, 0, 0]
    w1 = w1_ref[0, 0, 0]
    hn = (w0 + w1) * h_ref[0] + w0 * o0_ref[0] + w1 * o1_ref[0]
    h_out[0] = hn
    hbar = jnp.mean(hn, axis=0, keepdims=True)
    log_ref[0] = jnp.dot(hbar, rw_ref[...], preferred_element_type=jnp.float32)


def _combine(pos0, pos1, h, outbuf, w0, w1, rw_next):
    grid_spec = pltpu.PrefetchScalarGridSpec(
        num_scalar_prefetch=2,
        grid=(_B,),
        in_specs=[
            pl.BlockSpec((1, _NP, _D), lambda b, p0, p1: (b, 0, 0)),
            pl.BlockSpec((1, _NP, _D), lambda b, p0, p1: (p0[b], 0, 0)),
            pl.BlockSpec((1, _NP, _D), lambda b, p0, p1: (p1[b], 0, 0)),
            pl.BlockSpec((1, 1, 1), lambda b, p0, p1: (b, 0, 0)),
            pl.BlockSpec((1, 1, 1), lambda b, p0, p1: (b, 0, 0)),
            pl.BlockSpec((_D, _M), lambda b, p0, p1: (0, 0)),
        ],
        out_specs=[
            pl.BlockSpec((1, _NP, _D), lambda b, p0, p1: (b, 0, 0)),
            pl.BlockSpec((1, 1, _M), lambda b, p0, p1: (b, 0, 0)),
        ],
    )
    return pl.pallas_call(
        _combine_body,
        grid_spec=grid_spec,
        out_shape=[
            jax.ShapeDtypeStruct((_B, _NP, _D), jnp.float32),
            jax.ShapeDtypeStruct((_B, 1, _M), jnp.float32),
        ],
    )(pos0, pos1, h, outbuf, outbuf, w0, w1, rw_next)


def _final_body(p0_ref, p1_ref, h_ref, o0_ref, o1_ref, w0_ref, w1_ref,
                hw_ref, log_ref):
    w0 = w0_ref[0, 0, 0]
    w1 = w1_ref[0, 0, 0]
    hn = (w0 + w1) * h_ref[0] + w0 * o0_ref[0] + w1 * o1_ref[0]
    hbar = jnp.mean(hn, axis=0, keepdims=True)
    log_ref[0] = jnp.dot(hbar, hw_ref[...], preferred_element_type=jnp.float32)


def _final_combine(pos0, pos1, h, outbuf, w0, w1, head_W):
    grid_spec = pltpu.PrefetchScalarGridSpec(
        num_scalar_prefetch=2,
        grid=(_B,),
        in_specs=[
            pl.BlockSpec((1, _NP, _D), lambda b, p0, p1: (b, 0, 0)),
            pl.BlockSpec((1, _NP, _D), lambda b, p0, p1: (p0[b], 0, 0)),
            pl.BlockSpec((1, _NP, _D), lambda b, p0, p1: (p1[b], 0, 0)),
            pl.BlockSpec((1, 1, 1), lambda b, p0, p1: (b, 0, 0)),
            pl.BlockSpec((1, 1, 1), lambda b, p0, p1: (b, 0, 0)),
            pl.BlockSpec((_D, _NCLS), lambda b, p0, p1: (0, 0)),
        ],
        out_specs=pl.BlockSpec((1, 1, _NCLS), lambda b, p0, p1: (b, 0, 0)),
    )
    return pl.pallas_call(
        _final_body,
        grid_spec=grid_spec,
        out_shape=jax.ShapeDtypeStruct((_B, 1, _NCLS), jnp.float32),
    )(pos0, pos1, h, outbuf, outbuf, w0, w1, head_W)


def _route(logits3):
    logits = logits3[:, 0, :]
    probs = jax.nn.softmax(logits, axis=-1)
    top_vals, top_idx = jax.lax.top_k(probs, _K)
    sw = top_vals / (jnp.sum(top_vals, axis=-1, keepdims=True) + 1e-6)
    e_flat = top_idx.reshape(_A).astype(jnp.int32)
    order = jnp.argsort(e_flat).astype(jnp.int32)
    b_sorted = (order // _K).astype(jnp.int32)
    e_sorted = e_flat[order]
    inv = jnp.zeros((_A,), jnp.int32).at[order].set(
        jnp.arange(_A, dtype=jnp.int32))
    pos0 = inv[0::2]
    pos1 = inv[1::2]
    w0 = sw[:, 0].reshape(_B, 1, 1)
    w1 = sw[:, 1].reshape(_B, 1, 1)
    return b_sorted, e_sorted, pos0, pos1, w0, w1


@functools.partial(jax.jit)
def kernel(x, patch_W, patch_b, pos_embed, router_W, ln_g, ln_b,
           W1, b1, W2, b2, head_W):
    g = _IMG // _PATCH
    xp = x.reshape(_B, _C_IN, g, _PATCH, g, _PATCH)
    xp = xp.transpose(0, 2, 4, 1, 3, 5).reshape(_B, _NP, _CPP)
    patch_b2 = patch_b.reshape(1, _D)
    ln_g3 = ln_g.reshape(_M, 1, _D)
    ln_b3 = ln_b.reshape(_M, 1, _D)
    b13 = b1.reshape(_M, 1, _D_FF).astype(jnp.bfloat16)
    b23 = b2.reshape(_M, 1, _D)
    W1c = W1.astype(jnp.bfloat16)
    W2c = W2.astype(jnp.bfloat16)

    h, logits = _patch_embed(xp, patch_W, patch_b2, pos_embed, router_W[0])

    for hop in range(_HOPS):
        b_sorted, e_sorted, pos0, pos1, w0, w1 = _route(logits)
        outbuf = _expert_mlp(b_sorted, e_sorted, h, ln_g3, ln_b3, W1c, b13,
                             W2c, b23)
        if hop < _HOPS - 1:
            h, logits = _combine(pos0, pos1, h, outbuf, w0, w1,
                                 router_W[hop + 1])
        else:
            out = _final_combine(pos0, pos1, h, outbuf, w0, w1, head_W)
    return out.reshape(_B, _NCLS)


# speedup vs baseline: 3.4138x; 1.0488x over previous
import functools

import jax
import jax.numpy as jnp
from jax.experimental import pallas as pl
from jax.experimental.pallas import tpu as pltpu

_B = 64
_C_IN = 3
_IMG = 224
_PATCH = 16
_NP = (_IMG // _PATCH) ** 2
_D = 768
_D_FF = 3072
_M = 8
_K = 2
_HOPS = 4
_NCLS = 1000
_CPP = _C_IN * _PATCH * _PATCH
_A = _B * _K
_EPS = 1e-6


def _patch_body(xp_ref, w_ref, b_ref, pos_ref, rw_ref, h_ref, log_ref):
    t = xp_ref[0]
    hb = jnp.dot(t, w_ref[...], preferred_element_type=jnp.float32)
    hb = hb + b_ref[0] + pos_ref[0]
    h_ref[0] = hb
    hbar = jnp.mean(hb, axis=0, keepdims=True)
    log_ref[0] = jnp.dot(hbar, rw_ref[...], preferred_element_type=jnp.float32)


def _patch_embed(xp, patch_W, patch_b2, pos_embed, rw0):
    return pl.pallas_call(
        _patch_body,
        grid=(_B,),
        in_specs=[
            pl.BlockSpec((1, _NP, _CPP), lambda b: (b, 0, 0)),
            pl.BlockSpec((_CPP, _D), lambda b: (0, 0)),
            pl.BlockSpec((1, _D), lambda b: (0, 0)),
            pl.BlockSpec((1, _NP, _D), lambda b: (0, 0, 0)),
            pl.BlockSpec((_D, _M), lambda b: (0, 0)),
        ],
        out_specs=[
            pl.BlockSpec((1, _NP, _D), lambda b: (b, 0, 0)),
            pl.BlockSpec((1, 1, _M), lambda b: (b, 0, 0)),
        ],
        out_shape=[
            jax.ShapeDtypeStruct((_B, _NP, _D), jnp.float32),
            jax.ShapeDtypeStruct((_B, 1, _M), jnp.float32),
        ],
    )(xp, patch_W, patch_b2, pos_embed, rw0)


_C = 4
_PP = _A + _M * (_C - 1)
_NCH = _PP // _C


def _expert_body(bs_ref, ec_ref, af_ref, h0_ref, h1_ref, h2_ref, h3_ref,
                 lg_ref, lb_ref, w1_ref, b1_ref, w2_ref, b2_ref, out_ref):
    c = pl.program_id(0)

    @pl.when(af_ref[c] != 0)
    def _():
        for i, href in enumerate((h0_ref, h1_ref, h2_ref, h3_ref)):
            x = href[0]
            mu = jnp.mean(x, axis=-1, keepdims=True)
            xc = x - mu
            var = jnp.mean(xc * xc, axis=-1, keepdims=True)
            xln = xc * jax.lax.rsqrt(var + _EPS) * lg_ref[0] + lb_ref[0]
            h1 = jnp.dot(xln.astype(jnp.bfloat16), w1_ref[0],
                         preferred_element_type=jnp.float32)
            h1 = jax.nn.gelu((h1 + b1_ref[0]).astype(jnp.bfloat16))
            y = jnp.dot(h1, w2_ref[0],
                        preferred_element_type=jnp.float32) + b2_ref[0]
            out_ref[i] = y.astype(jnp.bfloat16)


def _expert_mlp(bs_pad, ec, af, h, ln_g3, ln_b3, W1, b13, W2, b23):
    hspec = [
        pl.BlockSpec((1, _NP, _D),
                     functools.partial(
                         lambda i, c, bs, ec, af: (bs[_C * c + i], 0, 0), i))
        for i in range(_C)
    ]
    grid_spec = pltpu.PrefetchScalarGridSpec(
        num_scalar_prefetch=3,
        grid=(_NCH,),
        in_specs=hspec + [
            pl.BlockSpec((1, 1, _D), lambda c, bs, ec, af: (ec[c], 0, 0)),
            pl.BlockSpec((1, 1, _D), lambda c, bs, ec, af: (ec[c], 0, 0)),
            pl.BlockSpec((1, _D, _D_FF), lambda c, bs, ec, af: (ec[c], 0, 0)),
            pl.BlockSpec((1, 1, _D_FF), lambda c, bs, ec, af: (ec[c], 0, 0)),
            pl.BlockSpec((1, _D_FF, _D), lambda c, bs, ec, af: (ec[c], 0, 0)),
            pl.BlockSpec((1, 1, _D), lambda c, bs, ec, af: (ec[c], 0, 0)),
        ],
        out_specs=pl.BlockSpec((_C, _NP, _D), lambda c, bs, ec, af: (c, 0, 0)),
    )
    return pl.pallas_call(
        _expert_body,
        grid_spec=grid_spec,
        out_shape=jax.ShapeDtypeStruct((_PP, _NP, _D), jnp.bfloat16),
    )(bs_pad, ec, af, h, h, h, h, ln_g3, ln_b3, W1, b13, W2, b23)


def _combine_body(p0_ref, p1_ref, h_ref, o0_ref, o1_ref, w0_ref, w1_ref,
                  rw_ref, h_out, log_ref):
    w0 = w0_ref[0, 0, 0]
    w1 = w1_ref[0, 0, 0]
    hn = (w0 + w1) * h_ref[0] + w0 * o0_ref[0] + w1 * o1_ref[0]
    h_out[0] = hn
    hbar = jnp.mean(hn, axis=0, keepdims=True)
    log_ref[0] = jnp.dot(hbar, rw_ref[...], preferred_element_type=jnp.float32)


def _combine(pos0, pos1, h, outbuf, w0, w1, rw_next):
    grid_spec = pltpu.PrefetchScalarGridSpec(
        num_scalar_prefetch=2,
        grid=(_B,),
        in_specs=[
            pl.BlockSpec((1, _NP, _D), lambda b, p0, p1: (b, 0, 0)),
            pl.BlockSpec((1, _NP, _D), lambda b, p0, p1: (p0[b], 0, 0)),
            pl.BlockSpec((1, _NP, _D), lambda b, p0, p1: (p1[b], 0, 0)),
            pl.BlockSpec((1, 1, 1), lambda b, p0, p1: (b, 0, 0)),
            pl.BlockSpec((1, 1, 1), lambda b, p0, p1: (b, 0, 0)),
            pl.BlockSpec((_D, _M), lambda b, p0, p1: (0, 0)),
        ],
        out_specs=[
            pl.BlockSpec((1, _NP, _D), lambda b, p0, p1: (b, 0, 0)),
            pl.BlockSpec((1, 1, _M), lambda b, p0, p1: (b, 0, 0)),
        ],
    )
    return pl.pallas_call(
        _combine_body,
        grid_spec=grid_spec,
        out_shape=[
            jax.ShapeDtypeStruct((_B, _NP, _D), jnp.float32),
            jax.ShapeDtypeStruct((_B, 1, _M), jnp.float32),
        ],
    )(pos0, pos1, h, outbuf, outbuf, w0, w1, rw_next)


def _final_body(p0_ref, p1_ref, h_ref, o0_ref, o1_ref, w0_ref, w1_ref,
                hw_ref, log_ref):
    w0 = w0_ref[0, 0, 0]
    w1 = w1_ref[0, 0, 0]
    hn = (w0 + w1) * h_ref[0] + w0 * o0_ref[0] + w1 * o1_ref[0]
    hbar = jnp.mean(hn, axis=0, keepdims=True)
    log_ref[0] = jnp.dot(hbar, hw_ref[...], preferred_element_type=jnp.float32)


def _final_combine(pos0, pos1, h, outbuf, w0, w1, head_W):
    grid_spec = pltpu.PrefetchScalarGridSpec(
        num_scalar_prefetch=2,
        grid=(_B,),
        in_specs=[
            pl.BlockSpec((1, _NP, _D), lambda b, p0, p1: (b, 0, 0)),
            pl.BlockSpec((1, _NP, _D), lambda b, p0, p1: (p0[b], 0, 0)),
            pl.BlockSpec((1, _NP, _D), lambda b, p0, p1: (p1[b], 0, 0)),
            pl.BlockSpec((1, 1, 1), lambda b, p0, p1: (b, 0, 0)),
            pl.BlockSpec((1, 1, 1), lambda b, p0, p1: (b, 0, 0)),
            pl.BlockSpec((_D, _NCLS), lambda b, p0, p1: (0, 0)),
        ],
        out_specs=pl.BlockSpec((1, 1, _NCLS), lambda b, p0, p1: (b, 0, 0)),
    )
    return pl.pallas_call(
        _final_body,
        grid_spec=grid_spec,
        out_shape=jax.ShapeDtypeStruct((_B, 1, _NCLS), jnp.float32),
    )(pos0, pos1, h, outbuf, outbuf, w0, w1, head_W)


def _route(logits3):
    logits = logits3[:, 0, :]
    probs = jax.nn.softmax(logits, axis=-1)
    top_vals, top_idx = jax.lax.top_k(probs, _K)
    sw = top_vals / (jnp.sum(top_vals, axis=-1, keepdims=True) + 1e-6)
    e_flat = top_idx.reshape(_A).astype(jnp.int32)
    order = jnp.argsort(e_flat).astype(jnp.int32)
    b_sorted = (order // _K).astype(jnp.int32)
    e_sorted = e_flat[order]

    counts = jnp.bincount(e_flat, length=_M)
    pc = ((counts + _C - 1) // _C) * _C
    cum_pc = jnp.cumsum(pc)
    start = (cum_pc - pc).astype(jnp.int32)
    off = (jnp.cumsum(counts) - counts).astype(jnp.int32)
    ii = jnp.arange(_A, dtype=jnp.int32)
    pad_pos = (start[e_sorted] + (ii - off[e_sorted])).astype(jnp.int32)

    bs_pad = jnp.zeros((_PP,), jnp.int32).at[pad_pos].set(b_sorted)
    cb = (cum_pc // _C).astype(jnp.int32)
    cidx = jnp.arange(_NCH, dtype=jnp.int32)
    ec = jnp.clip(jnp.searchsorted(cb, cidx, side='right'),
                  0, _M - 1).astype(jnp.int32)
    af = (cidx < cb[_M - 1]).astype(jnp.int32)

    inv = jnp.zeros((_A,), jnp.int32).at[order].set(pad_pos)
    pos0 = inv[0::2]
    pos1 = inv[1::2]
    w0 = sw[:, 0].reshape(_B, 1, 1)
    w1 = sw[:, 1].reshape(_B, 1, 1)
    return bs_pad, ec, af, pos0, pos1, w0, w1


@functools.partial(jax.jit)
def kernel(x, patch_W, patch_b, pos_embed, router_W, ln_g, ln_b,
           W1, b1, W2, b2, head_W):
    g = _IMG // _PATCH
    xp = x.reshape(_B, _C_IN, g, _PATCH, g, _PATCH)
    xp = xp.transpose(0, 2, 4, 1, 3, 5).reshape(_B, _NP, _CPP)
    patch_b2 = patch_b.reshape(1, _D)
    ln_g3 = ln_g.reshape(_M, 1, _D)
    ln_b3 = ln_b.reshape(_M, 1, _D)
    b13 = b1.reshape(_M, 1, _D_FF).astype(jnp.bfloat16)
    b23 = b2.reshape(_M, 1, _D)
    W1c = W1.astype(jnp.bfloat16)
    W2c = W2.astype(jnp.bfloat16)

    h, logits = _patch_embed(xp, patch_W, patch_b2, pos_embed, router_W[0])

    for hop in range(_HOPS):
        bs_pad, ec, af, pos0, pos1, w0, w1 = _route(logits)
        outbuf = _expert_mlp(bs_pad, ec, af, h, ln_g3, ln_b3, W1c, b13,
                             W2c, b23)
        if hop < _HOPS - 1:
            h, logits = _combine(pos0, pos1, h, outbuf, w0, w1,
                                 router_W[hop + 1])
        else:
            out = _final_combine(pos0, pos1, h, outbuf, w0, w1, head_W)
    return out.reshape(_B, _NCLS)
